# initial kernel scaffold (unmeasured)
import jax
import jax.numpy as jnp
from jax import lax
from jax.experimental import pallas as pl
from jax.experimental.pallas import tpu as pltpu

N_DEV = 4
WINDOW = 128


def kernel(x, Wq, K_ext, V_ext, Wo):
    B_loc, Sq, D = x.shape
    _, Hdim = Wq.shape
    Bg, Skv, Hq, Dh = K_ext.shape
    H_loc = Hdim // Dh

    me = lax.axis_index("i")

    k_loc = lax.dynamic_slice_in_dim(K_ext, B_loc * me, B_loc, axis=0)
    v_loc = lax.dynamic_slice_in_dim(V_ext, B_loc * me, B_loc, axis=0)

    grp = jnp.mod(me - jnp.arange(N_DEV), N_DEV)
    head_idx = (grp[:, None] * H_loc + jnp.arange(H_loc)[None, :]).reshape(-1)
    k_arr = jnp.take(k_loc.transpose(0, 2, 1, 3), head_idx, axis=1)
    v_arr = jnp.take(v_loc.transpose(0, 2, 1, 3), head_idx, axis=1)

    def body(x_ref, wq_ref, k_ref, v_ref, wo_ref, out_ref,
             wq_comm, wo_comm, wq_send, wq_recv, wo_send, wo_recv):
        my = lax.axis_index("i")
        left = lax.rem(my + (N_DEV - 1), N_DEV)
        right = lax.rem(my + 1, N_DEV)

        barrier = pltpu.get_barrier_semaphore()
        for nbr in (left, right):
            pl.semaphore_signal(
                barrier, inc=1,
                device_id=(nbr,), device_id_type=pl.DeviceIdType.MESH,
            )
        pl.semaphore_wait(barrier, 2)

        qi = lax.broadcasted_iota(jnp.int32, (Sq, Skv), 0)
        ki = lax.broadcasted_iota(jnp.int32, (Sq, Skv), 1)
        win = jnp.abs(qi - ki) <= WINDOW

        def compute_step(wq_w, wo_w, h):
            for b in range(B_loc):
                q_full = jnp.dot(x_ref[b], wq_w,
                                 preferred_element_type=jnp.float32)
                ctx_cols = []
                for hl in range(H_loc):
                    gh = h * H_loc + hl
                    q = q_full[:, hl * Dh:(hl + 1) * Dh]
                    k = k_ref[b, gh]
                    v = v_ref[b, gh]
                    s = lax.dot_general(
                        q, k, (((1,), (1,)), ((), ())),
                        preferred_element_type=jnp.float32) * 0.125
                    s = jnp.where(win, s, jnp.float32(-1e9))
                    m = jnp.max(s, axis=1, keepdims=True)
                    w = jnp.exp(s - m)
                    w = w / jnp.sum(w, axis=1, keepdims=True)
                    ctx_cols.append(
                        jnp.dot(w, v, preferred_element_type=jnp.float32))
                ctx = jnp.concatenate(ctx_cols, axis=1)
                contrib = jnp.dot(ctx, wo_w,
                                  preferred_element_type=jnp.float32)
                if h == 0:
                    out_ref[b] = contrib
                else:
                    out_ref[b] = out_ref[b] + contrib

        for hop in range(N_DEV - 1):
            src_wq = wq_ref if hop == 0 else wq_comm.at[hop - 1]
            src_wo = wo_ref if hop == 0 else wo_comm.at[hop - 1]
            r_wq = pltpu.make_async_remote_copy(
                src_ref=src_wq, dst_ref=wq_comm.at[hop],
                send_sem=wq_send.at[hop], recv_sem=wq_recv.at[hop],
                device_id=(right,), device_id_type=pl.DeviceIdType.MESH)
            r_wo = pltpu.make_async_remote_copy(
                src_ref=src_wo, dst_ref=wo_comm.at[hop],
                send_sem=wo_send.at[hop], recv_sem=wo_recv.at[hop],
                device_id=(right,), device_id_type=pl.DeviceIdType.MESH)
            r_wq.start()
            r_wo.start()
            if hop == 0:
                compute_step(wq_ref[...], wo_ref[...], 0)
            else:
                compute_step(wq_comm[hop - 1], wo_comm[hop - 1], hop)
            r_wq.wait()
            r_wo.wait()
        compute_step(wq_comm[N_DEV - 2], wo_comm[N_DEV - 2], N_DEV - 1)

    return pl.pallas_call(
        body,
        out_shape=jax.ShapeDtypeStruct((B_loc, Sq, D), jnp.float32),
        in_specs=[pl.BlockSpec(memory_space=pltpu.VMEM)] * 5,
        out_specs=pl.BlockSpec(memory_space=pltpu.VMEM),
        scratch_shapes=[
            pltpu.VMEM((N_DEV - 1, D, Hdim), jnp.float32),
            pltpu.VMEM((N_DEV - 1, Hdim, D), jnp.float32),
            pltpu.SemaphoreType.DMA((N_DEV - 1,)),
            pltpu.SemaphoreType.DMA((N_DEV - 1,)),
            pltpu.SemaphoreType.DMA((N_DEV - 1,)),
            pltpu.SemaphoreType.DMA((N_DEV - 1,)),
        ],
        compiler_params=pltpu.CompilerParams(collective_id=0),
    )(x, Wq, k_arr, v_arr, Wo)


# baseline (device time: 188273 ns/iter reference)
import jax
import jax.numpy as jnp
from jax import lax
from jax.experimental import pallas as pl
from jax.experimental.pallas import tpu as pltpu

N_DEV = 4
WINDOW = 128


def kernel(x, Wq, K_ext, V_ext, Wo):
    B_loc, Sq, D = x.shape
    _, Hdim = Wq.shape
    Bg, Skv, Hq, Dh = K_ext.shape
    H_loc = Hdim // Dh

    me = lax.axis_index("i")

    k_loc = lax.dynamic_slice_in_dim(K_ext, B_loc * me, B_loc, axis=0)
    v_loc = lax.dynamic_slice_in_dim(V_ext, B_loc * me, B_loc, axis=0)

    grp = jnp.mod(me - jnp.arange(N_DEV), N_DEV)
    head_idx = (grp[:, None] * H_loc + jnp.arange(H_loc)[None, :]).reshape(-1)
    k_arr = jnp.take(k_loc.transpose(0, 2, 1, 3), head_idx, axis=1)
    v_arr = jnp.take(v_loc.transpose(0, 2, 1, 3), head_idx, axis=1)

    def body(x_ref, wq_ref, k_ref, v_ref, wo_ref, out_ref,
             wq_comm, wo_comm, wq_send, wq_recv, wo_send, wo_recv):
        my = lax.axis_index("i")
        left = lax.rem(my + (N_DEV - 1), N_DEV)
        right = lax.rem(my + 1, N_DEV)

        barrier = pltpu.get_barrier_semaphore()
        for nbr in (left, right):
            pl.semaphore_signal(
                barrier, inc=1,
                device_id=(nbr,), device_id_type=pl.DeviceIdType.MESH,
            )
        pl.semaphore_wait(barrier, 2)

        qi = lax.broadcasted_iota(jnp.int32, (Sq, Skv), 0)
        ki = lax.broadcasted_iota(jnp.int32, (Sq, Skv), 1)
        win = jnp.abs(qi - ki) <= WINDOW

        def compute_step(wq_w, wo_w, h):
            for b in range(B_loc):
                q_full = jnp.dot(x_ref[b], wq_w,
                                 preferred_element_type=jnp.float32)
                ctx_cols = []
                for hl in range(H_loc):
                    gh = h * H_loc + hl
                    q = q_full[:, hl * Dh:(hl + 1) * Dh]
                    k = k_ref[b, gh]
                    v = v_ref[b, gh]
                    s = lax.dot_general(
                        q, k, (((1,), (1,)), ((), ())),
                        preferred_element_type=jnp.float32) * 0.125
                    s = jnp.where(win, s, jnp.float32(-1e9))
                    m = jnp.max(s, axis=1, keepdims=True)
                    w = jnp.exp(s - m)
                    w = w / jnp.sum(w, axis=1, keepdims=True)
                    ctx_cols.append(
                        jnp.dot(w, v, preferred_element_type=jnp.float32))
                ctx = jnp.concatenate(ctx_cols, axis=1)
                contrib = jnp.dot(ctx, wo_w,
                                  preferred_element_type=jnp.float32)
                if h == 0:
                    out_ref[b] = contrib
                else:
                    out_ref[b] = out_ref[b] + contrib

        for hop in range(N_DEV - 1):
            src_wq = wq_ref if hop == 0 else wq_comm.at[hop - 1]
            src_wo = wo_ref if hop == 0 else wo_comm.at[hop - 1]
            r_wq = pltpu.make_async_remote_copy(
                src_ref=src_wq, dst_ref=wq_comm.at[hop],
                send_sem=wq_send.at[hop], recv_sem=wq_recv.at[hop],
                device_id=(right,), device_id_type=pl.DeviceIdType.MESH)
            r_wo = pltpu.make_async_remote_copy(
                src_ref=src_wo, dst_ref=wo_comm.at[hop],
                send_sem=wo_send.at[hop], recv_sem=wo_recv.at[hop],
                device_id=(right,), device_id_type=pl.DeviceIdType.MESH)
            r_wq.start()
            r_wo.start()
            if hop == 0:
                compute_step(wq_ref[...], wo_ref[...], 0)
            else:
                compute_step(wq_comm[hop - 1], wo_comm[hop - 1], hop)
            r_wq.wait()
            r_wo.wait()
        compute_step(wq_comm[N_DEV - 2], wo_comm[N_DEV - 2], N_DEV - 1)

    return pl.pallas_call(
        body,
        out_shape=jax.ShapeDtypeStruct((B_loc, Sq, D), jnp.float32),
        in_specs=[pl.BlockSpec(memory_space=pltpu.VMEM)] * 5,
        out_specs=pl.BlockSpec(memory_space=pltpu.VMEM),
        scratch_shapes=[
            pltpu.VMEM((N_DEV - 1, D, Hdim), jnp.float32),
            pltpu.VMEM((N_DEV - 1, Hdim, D), jnp.float32),
            pltpu.SemaphoreType.DMA((N_DEV - 1,)),
            pltpu.SemaphoreType.DMA((N_DEV - 1,)),
            pltpu.SemaphoreType.DMA((N_DEV - 1,)),
            pltpu.SemaphoreType.DMA((N_DEV - 1,)),
        ],
        compiler_params=pltpu.CompilerParams(
            collective_id=0, vmem_limit_bytes=100 * 1024 * 1024),
    )(x, Wq, k_arr, v_arr, Wo)


# device time: 102814 ns/iter; 1.8312x vs baseline; 1.8312x over previous
import jax
import jax.numpy as jnp
from jax import lax
from jax.experimental import pallas as pl
from jax.experimental.pallas import tpu as pltpu

N_DEV = 4
WINDOW = 128
CDT = jnp.bfloat16


def kernel(x, Wq, K_ext, V_ext, Wo):
    B_loc, Sq, D = x.shape
    _, Hdim = Wq.shape
    Bg, Skv, Hq, Dh = K_ext.shape
    H_loc = Hdim // Dh

    me = lax.axis_index("i")

    k_arr = lax.dynamic_slice_in_dim(
        K_ext, B_loc * me, B_loc, axis=0).astype(CDT).transpose(0, 2, 1, 3)
    v_arr = lax.dynamic_slice_in_dim(
        V_ext, B_loc * me, B_loc, axis=0).astype(CDT).transpose(0, 2, 1, 3)

    def body(x_ref, wq_ref, k_ref, v_ref, wo_ref, out_ref,
             wq_comm, wo_comm, wq_send, wq_recv, wo_send, wo_recv):
        my = lax.axis_index("i")
        left = lax.rem(my + (N_DEV - 1), N_DEV)
        right = lax.rem(my + 1, N_DEV)

        barrier = pltpu.get_barrier_semaphore()
        for nbr in (left, right):
            pl.semaphore_signal(
                barrier, inc=1,
                device_id=(nbr,), device_id_type=pl.DeviceIdType.MESH,
            )
        pl.semaphore_wait(barrier, 2)

        qi = lax.broadcasted_iota(jnp.int32, (Sq, Skv), 0)
        ki = lax.broadcasted_iota(jnp.int32, (Sq, Skv), 1)
        win = jnp.abs(qi - ki) <= WINDOW

        def compute_step(wq_w, wo_w, h):
            grp_base = lax.rem(my - h + N_DEV, N_DEV) * H_loc
            for b in range(B_loc):
                q_full = jnp.dot(x_ref[b], wq_w,
                                 preferred_element_type=jnp.float32
                                 ).astype(CDT)
                ctx_cols = []
                for hl in range(H_loc):
                    gh = grp_base + hl
                    q = q_full[:, hl * Dh:(hl + 1) * Dh]
                    k = k_ref[b, gh]
                    v = v_ref[b, gh]
                    s = lax.dot_general(
                        q, k, (((1,), (1,)), ((), ())),
                        preferred_element_type=jnp.float32) * 0.125
                    s = jnp.where(win, s, jnp.float32(-1e9))
                    m = jnp.max(s, axis=1, keepdims=True)
                    w = jnp.exp(s - m)
                    w = (w / jnp.sum(w, axis=1, keepdims=True)).astype(CDT)
                    ctx_cols.append(
                        jnp.dot(w, v, preferred_element_type=jnp.float32))
                ctx = jnp.concatenate(ctx_cols, axis=1).astype(CDT)
                contrib = jnp.dot(ctx, wo_w,
                                  preferred_element_type=jnp.float32)
                if h == 0:
                    out_ref[b] = contrib
                else:
                    out_ref[b] = out_ref[b] + contrib

        for hop in range(N_DEV - 1):
            src_wq = wq_ref if hop == 0 else wq_comm.at[hop - 1]
            src_wo = wo_ref if hop == 0 else wo_comm.at[hop - 1]
            r_wq = pltpu.make_async_remote_copy(
                src_ref=src_wq, dst_ref=wq_comm.at[hop],
                send_sem=wq_send.at[hop], recv_sem=wq_recv.at[hop],
                device_id=(right,), device_id_type=pl.DeviceIdType.MESH)
            r_wo = pltpu.make_async_remote_copy(
                src_ref=src_wo, dst_ref=wo_comm.at[hop],
                send_sem=wo_send.at[hop], recv_sem=wo_recv.at[hop],
                device_id=(right,), device_id_type=pl.DeviceIdType.MESH)
            r_wq.start()
            r_wo.start()
            if hop == 0:
                compute_step(wq_ref[...], wo_ref[...], 0)
            else:
                compute_step(wq_comm[hop - 1], wo_comm[hop - 1], hop)
            r_wq.wait()
            r_wo.wait()
        compute_step(wq_comm[N_DEV - 2], wo_comm[N_DEV - 2], N_DEV - 1)

    return pl.pallas_call(
        body,
        out_shape=jax.ShapeDtypeStruct((B_loc, Sq, D), jnp.float32),
        in_specs=[pl.BlockSpec(memory_space=pltpu.VMEM)] * 5,
        out_specs=pl.BlockSpec(memory_space=pltpu.VMEM),
        scratch_shapes=[
            pltpu.VMEM((N_DEV - 1, D, Hdim), CDT),
            pltpu.VMEM((N_DEV - 1, Hdim, D), CDT),
            pltpu.SemaphoreType.DMA((N_DEV - 1,)),
            pltpu.SemaphoreType.DMA((N_DEV - 1,)),
            pltpu.SemaphoreType.DMA((N_DEV - 1,)),
            pltpu.SemaphoreType.DMA((N_DEV - 1,)),
        ],
        compiler_params=pltpu.CompilerParams(
            collective_id=0, vmem_limit_bytes=100 * 1024 * 1024),
    )(x.astype(CDT), Wq.astype(CDT), k_arr, v_arr, Wo.astype(CDT))


# device time: 90773 ns/iter; 2.0741x vs baseline; 1.1326x over previous
import jax
import jax.numpy as jnp
from jax import lax
from jax.experimental import pallas as pl
from jax.experimental.pallas import tpu as pltpu

N_DEV = 4
WINDOW = 128
CDT = jnp.bfloat16


def kernel(x, Wq, K_ext, V_ext, Wo):
    B_loc, Sq, D = x.shape
    _, Hdim = Wq.shape
    Bg, Skv, Hq, Dh = K_ext.shape
    H_loc = Hdim // Dh

    me = lax.axis_index("i")

    k_arr = lax.dynamic_slice_in_dim(
        K_ext, B_loc * me, B_loc, axis=0).astype(CDT).transpose(0, 2, 1, 3)
    v_arr = lax.dynamic_slice_in_dim(
        V_ext, B_loc * me, B_loc, axis=0).astype(CDT).transpose(0, 2, 1, 3)

    def body(x_ref, wq_ref, k_ref, v_ref, wo_ref, out_ref,
             wq_comm, wo_comm, wq_send, wq_recv, wo_send, wo_recv):
        my = lax.axis_index("i")

        barrier = pltpu.get_barrier_semaphore()
        for d in range(1, N_DEV):
            pl.semaphore_signal(
                barrier, inc=1,
                device_id=(lax.rem(my + d, N_DEV),),
                device_id_type=pl.DeviceIdType.MESH,
            )
        pl.semaphore_wait(barrier, N_DEV - 1)

        qi = lax.broadcasted_iota(jnp.int32, (Sq, Skv), 0)
        ki = lax.broadcasted_iota(jnp.int32, (Sq, Skv), 1)
        win = jnp.abs(qi - ki) <= WINDOW

        def compute_step(wq_w, wo_w, h):
            grp_base = lax.rem(my - h + N_DEV, N_DEV) * H_loc
            for b in range(B_loc):
                q_full = jnp.dot(x_ref[b], wq_w,
                                 preferred_element_type=jnp.float32
                                 ).astype(CDT)
                ctx_cols = []
                for hl in range(H_loc):
                    gh = grp_base + hl
                    q = q_full[:, hl * Dh:(hl + 1) * Dh]
                    k = k_ref[b, gh]
                    v = v_ref[b, gh]
                    s = lax.dot_general(
                        q, k, (((1,), (1,)), ((), ())),
                        preferred_element_type=jnp.float32) * 0.125
                    s = jnp.where(win, s, jnp.float32(-1e9))
                    m = jnp.max(s, axis=1, keepdims=True)
                    w = jnp.exp(s - m)
                    w = (w / jnp.sum(w, axis=1, keepdims=True)).astype(CDT)
                    ctx_cols.append(
                        jnp.dot(w, v, preferred_element_type=jnp.float32))
                ctx = jnp.concatenate(ctx_cols, axis=1).astype(CDT)
                contrib = jnp.dot(ctx, wo_w,
                                  preferred_element_type=jnp.float32)
                if h == 0:
                    out_ref[b] = contrib
                else:
                    out_ref[b] = out_ref[b] + contrib

        sends = []
        for d in range(1, N_DEV):
            dst = lax.rem(my + d, N_DEV)
            r_wq = pltpu.make_async_remote_copy(
                src_ref=wq_ref, dst_ref=wq_comm.at[d - 1],
                send_sem=wq_send.at[d - 1], recv_sem=wq_recv.at[d - 1],
                device_id=(dst,), device_id_type=pl.DeviceIdType.MESH)
            r_wo = pltpu.make_async_remote_copy(
                src_ref=wo_ref, dst_ref=wo_comm.at[d - 1],
                send_sem=wo_send.at[d - 1], recv_sem=wo_recv.at[d - 1],
                device_id=(dst,), device_id_type=pl.DeviceIdType.MESH)
            r_wq.start()
            r_wo.start()
            sends += [r_wq, r_wo]

        compute_step(wq_ref[...], wo_ref[...], 0)
        for h in range(1, N_DEV):
            pltpu.make_async_remote_copy(
                src_ref=wq_ref, dst_ref=wq_comm.at[h - 1],
                send_sem=wq_send.at[h - 1], recv_sem=wq_recv.at[h - 1],
                device_id=(my,), device_id_type=pl.DeviceIdType.MESH,
            ).wait_recv()
            pltpu.make_async_remote_copy(
                src_ref=wo_ref, dst_ref=wo_comm.at[h - 1],
                send_sem=wo_send.at[h - 1], recv_sem=wo_recv.at[h - 1],
                device_id=(my,), device_id_type=pl.DeviceIdType.MESH,
            ).wait_recv()
            compute_step(wq_comm[h - 1], wo_comm[h - 1], h)
        for r in sends:
            r.wait_send()

    return pl.pallas_call(
        body,
        out_shape=jax.ShapeDtypeStruct((B_loc, Sq, D), jnp.float32),
        in_specs=[pl.BlockSpec(memory_space=pltpu.VMEM)] * 5,
        out_specs=pl.BlockSpec(memory_space=pltpu.VMEM),
        scratch_shapes=[
            pltpu.VMEM((N_DEV - 1, D, Hdim), CDT),
            pltpu.VMEM((N_DEV - 1, Hdim, D), CDT),
            pltpu.SemaphoreType.DMA((N_DEV - 1,)),
            pltpu.SemaphoreType.DMA((N_DEV - 1,)),
            pltpu.SemaphoreType.DMA((N_DEV - 1,)),
            pltpu.SemaphoreType.DMA((N_DEV - 1,)),
        ],
        compiler_params=pltpu.CompilerParams(
            collective_id=0, vmem_limit_bytes=100 * 1024 * 1024),
    )(x.astype(CDT), Wq.astype(CDT), k_arr, v_arr, Wo.astype(CDT))
